# Initial kernel scaffold; baseline (speedup 1.0000x reference)
import jax
import jax.numpy as jnp
from jax.experimental import pallas as pl


def kernel(x, edge_index, Wl1, Wr1, b1, Wl2, Wr2, b2, Wl3, Wr3, b3):
    raise NotImplementedError("write your pallas kernel here")


# speedup vs baseline: 3.6298x; 3.6298x over previous
import functools

import jax
import jax.numpy as jnp
from jax import lax
from jax.experimental import pallas as pl
from jax.experimental.pallas import tpu as pltpu
from jax.experimental.pallas import tpu_sc as plsc

_N = 50000
_E = 800000
_NC = 2
_NS = 16
_CH = 128
_EPAD = 802816
_NA = 196
_NB = 392
_NACC = 51200
_BN = 2000


def _seg_sum(n_chunks):
    mesh = plsc.VectorSubcoreMesh(core_axis_name="c", subcore_axis_name="s")
    zit = (_NACC // _NS) // _CH

    @functools.partial(
        pl.kernel,
        out_type=jax.ShapeDtypeStruct((_NC, _NACC, 8), jnp.float32),
        mesh=mesh,
        scratch_types=[
            pltpu.VMEM((n_chunks, _CH), jnp.int32),
            pltpu.VMEM((n_chunks, _CH), jnp.int32),
            pltpu.VMEM((_CH, 8), jnp.float32),
            pltpu.VMEM_SHARED((_NACC, 8), jnp.float32),
            pltpu.SemaphoreType.DMA,
        ],
        compiler_params=pltpu.CompilerParams(use_tc_tiling_on_sc=False),
    )
    def k(src_hbm, dst_hbm, table_hbm, zeros_hbm, out_hbm,
          src_v, dst_v, rows_v, acc, sem):
        c = lax.axis_index("c")
        s = lax.axis_index("s")

        pltpu.sync_copy(zeros_hbm, rows_v)
        z0 = s * (_NACC // _NS)

        def zbody(i, carry):
            pltpu.sync_copy(rows_v, acc.at[pl.ds(z0 + i * _CH, _CH)])
            return carry
        lax.fori_loop(0, zit, zbody, 0)
        plsc.subcore_barrier()

        pltpu.sync_copy(src_hbm.at[c, s], src_v)
        pltpu.sync_copy(dst_hbm.at[c, s], dst_v)

        def ebody(j, carry):
            pltpu.async_copy(table_hbm.at[src_v.at[j]], rows_v, sem).wait()
            pltpu.sync_copy(rows_v, acc.at[dst_v.at[j]], add=True)
            return carry
        lax.fori_loop(0, n_chunks, ebody, 0)
        plsc.subcore_barrier()

        def obody(i, carry):
            pltpu.sync_copy(acc.at[pl.ds(z0 + i * _CH, _CH)], rows_v)
            pltpu.sync_copy(rows_v, out_hbm.at[c, pl.ds(z0 + i * _CH, _CH)])
            return carry
        lax.fori_loop(0, zit, obody, 0)

    return k


_seg_edge = _seg_sum(_NA)
_seg_feat = _seg_sum(_NB)


def _tc1(parts, x_aug, wl, wr, b):
    def body(p_ref, x_ref, wl_ref, wr_ref, b_ref, h_ref, inv_ref):
        sums = p_ref[0] + p_ref[1]
        inv = 1.0 / jnp.maximum(sums[:, 5:6], 1.0)
        h = (sums * inv) @ wl_ref[...] + x_ref[...] @ wr_ref[...] + b_ref[...]
        h = jnp.maximum(h, 0.0)
        for q in range(8):
            h_ref[q] = h[:, 8 * q:8 * (q + 1)]
        inv_ref[...] = inv

    return pl.pallas_call(
        body,
        grid=(_N // _BN,),
        in_specs=[
            pl.BlockSpec((2, _BN, 8), lambda i: (0, i, 0)),
            pl.BlockSpec((_BN, 8), lambda i: (i, 0)),
            pl.BlockSpec((8, 64), lambda i: (0, 0)),
            pl.BlockSpec((8, 64), lambda i: (0, 0)),
            pl.BlockSpec((1, 64), lambda i: (0, 0)),
        ],
        out_specs=[
            pl.BlockSpec((8, _BN, 8), lambda i: (0, i, 0)),
            pl.BlockSpec((_BN, 1), lambda i: (i, 0)),
        ],
        out_shape=[
            jax.ShapeDtypeStruct((8, _N, 8), jnp.float32),
            jax.ShapeDtypeStruct((_N, 1), jnp.float32),
        ],
    )(parts, x_aug, wl, wr, b)


def _tc2(a0, a1, a2, a3, h_split, inv, wl2, wr2, b2, wl3, wr3, b3):
    def body(a0_ref, a1_ref, a2_ref, a3_ref, h_ref, inv_ref,
             wl2_ref, wr2_ref, b2_ref, wl3_ref, wr3_ref, b3_ref,
             p_ref, r_ref):
        blocks = []
        for a_ref in (a0_ref, a1_ref, a2_ref, a3_ref):
            blocks.append(a_ref[0])
            blocks.append(a_ref[1])
        agg = jnp.concatenate(blocks, axis=1) * inv_ref[...]
        h1 = jnp.concatenate([h_ref[q] for q in range(8)], axis=1)
        h2 = agg @ wl2_ref[...] + h1 @ wr2_ref[...] + b2_ref[...]
        h2 = jnp.maximum(h2, 0.0)
        p = h2 @ wl3_ref[...]
        for q in range(4):
            p_ref[q] = p[:, 8 * q:8 * (q + 1)]
        r_ref[...] = h2 @ wr3_ref[...] + b3_ref[...]

    aspec = pl.BlockSpec((2, _BN, 8), lambda i: (0, i, 0))
    return pl.pallas_call(
        body,
        grid=(_N // _BN,),
        in_specs=[
            aspec, aspec, aspec, aspec,
            pl.BlockSpec((8, _BN, 8), lambda i: (0, i, 0)),
            pl.BlockSpec((_BN, 1), lambda i: (i, 0)),
            pl.BlockSpec((64, 64), lambda i: (0, 0)),
            pl.BlockSpec((64, 64), lambda i: (0, 0)),
            pl.BlockSpec((1, 64), lambda i: (0, 0)),
            pl.BlockSpec((64, 32), lambda i: (0, 0)),
            pl.BlockSpec((64, 32), lambda i: (0, 0)),
            pl.BlockSpec((1, 32), lambda i: (0, 0)),
        ],
        out_specs=[
            pl.BlockSpec((4, _BN, 8), lambda i: (0, i, 0)),
            pl.BlockSpec((_BN, 32), lambda i: (i, 0)),
        ],
        out_shape=[
            jax.ShapeDtypeStruct((4, _N, 8), jnp.float32),
            jax.ShapeDtypeStruct((_N, 32), jnp.float32),
        ],
    )(a0, a1, a2, a3, h_split, inv, wl2, wr2, b2, wl3, wr3, b3)


def _tc3(g0, g1, inv, r):
    def body(g0_ref, g1_ref, inv_ref, r_ref, out_ref):
        agg = jnp.concatenate([g0_ref[0], g0_ref[1], g1_ref[0], g1_ref[1]],
                              axis=1)
        out_ref[...] = agg * inv_ref[...] + r_ref[...]

    gspec = pl.BlockSpec((2, _BN, 8), lambda i: (0, i, 0))
    return pl.pallas_call(
        body,
        grid=(_N // _BN,),
        in_specs=[
            gspec, gspec,
            pl.BlockSpec((_BN, 1), lambda i: (i, 0)),
            pl.BlockSpec((_BN, 32), lambda i: (i, 0)),
        ],
        out_specs=pl.BlockSpec((_BN, 32), lambda i: (i, 0)),
        out_shape=jax.ShapeDtypeStruct((_N, 32), jnp.float32),
    )(g0, g1, inv, r)


def kernel(x, edge_index, Wl1, Wr1, b1, Wl2, Wr2, b2, Wl3, Wr3, b3):
    src = edge_index[0]
    dst = edge_index[1]
    pad = _EPAD - _E
    srcp = jnp.concatenate([src, jnp.zeros((pad,), jnp.int32)])
    dstp = jnp.concatenate([dst, jnp.full((pad,), _N, jnp.int32)])

    srcA = srcp.reshape(_NC, _NS, _NA, _CH)
    dstA = dstp.reshape(_NC, _NS, _NA, _CH)
    srcS = srcp.reshape(1, _NS, _NB, _CH)
    srcF = [jnp.concatenate([srcS + (2 * q) * _N, srcS + (2 * q + 1) * _N])
            for q in range(4)]
    dstB = jnp.broadcast_to(dstp.reshape(1, _NS, _NB, _CH),
                            (_NC, _NS, _NB, _CH))

    ones = jnp.ones((_N, 1), jnp.float32)
    x_aug = jnp.concatenate([x, ones, jnp.zeros((_N, 2), jnp.float32)], axis=1)
    z8 = jnp.zeros((_CH, 8), jnp.float32)
    wpad = jnp.zeros((3, 64), jnp.float32)
    wl1p = jnp.concatenate([Wl1, wpad], axis=0)
    wr1p = jnp.concatenate([Wr1, wpad], axis=0)

    parts1 = _seg_edge(srcA, dstA, x_aug, z8)[:, :_N]
    h_split, inv = _tc1(parts1, x_aug, wl1p, wr1p, b1.reshape(1, 64))
    table2 = h_split.reshape(8 * _N, 8)
    agg2 = [_seg_feat(srcF[q], dstB, table2, z8)[:, :_N] for q in range(4)]
    p_split, r = _tc2(agg2[0], agg2[1], agg2[2], agg2[3], h_split, inv,
                      Wl2, Wr2, b2.reshape(1, 64), Wl3, Wr3, b3.reshape(1, 32))
    table3 = p_split.reshape(4 * _N, 8)
    agg3 = [_seg_feat(srcF[q], dstB, table3, z8)[:, :_N] for q in range(2)]
    return _tc3(agg3[0], agg3[1], inv, r)


# speedup vs baseline: 5.3285x; 1.4680x over previous
import functools

import jax
import jax.numpy as jnp
from jax import lax
from jax.experimental import pallas as pl
from jax.experimental.pallas import tpu as pltpu
from jax.experimental.pallas import tpu_sc as plsc

_N = 50000
_E = 800000
_NC = 2
_NS = 16
_CH = 128
_EPAD = 802816
_NA = 196
_NB = 392
_NACC = 51200
_BN = 2000


def _seg_sum(n_chunks):
    mesh = plsc.VectorSubcoreMesh(core_axis_name="c", subcore_axis_name="s")
    zit = (_NACC // _NS) // _CH

    @functools.partial(
        pl.kernel,
        out_type=jax.ShapeDtypeStruct((_NC, _NACC, 8), jnp.float32),
        mesh=mesh,
        scratch_types=[
            pltpu.VMEM((n_chunks, _CH), jnp.int32),
            pltpu.VMEM((n_chunks, _CH), jnp.int32),
            pltpu.VMEM((_CH, 8), jnp.float32),
            pltpu.VMEM((_CH, 8), jnp.float32),
            pltpu.VMEM_SHARED((_NACC, 8), jnp.float32),
            pltpu.SemaphoreType.DMA,
            pltpu.SemaphoreType.DMA,
        ],
        compiler_params=pltpu.CompilerParams(use_tc_tiling_on_sc=False),
    )
    def k(src_hbm, dst_hbm, table_hbm, zeros_hbm, out_hbm,
          src_v, dst_v, rows0, rows1, acc, sem0, sem1):
        c = lax.axis_index("c")
        s = lax.axis_index("s")

        pltpu.sync_copy(zeros_hbm, rows0)
        z0 = s * (_NACC // _NS)

        def zbody(i, carry):
            pltpu.sync_copy(rows0, acc.at[pl.ds(z0 + i * _CH, _CH)])
            return carry
        lax.fori_loop(0, zit, zbody, 0)
        plsc.subcore_barrier()

        pltpu.sync_copy(src_hbm.at[c, s], src_v)
        pltpu.sync_copy(dst_hbm.at[c, s], dst_v)

        bufs = (rows0, rows1)
        sems = (sem0, sem1)
        last = n_chunks - 1

        pltpu.async_copy(table_hbm.at[src_v.at[0]], rows0, sem0)

        def ebody(t, carry):
            for b in range(2):
                j = 2 * t + b
                jn = jnp.minimum(j + 1, last)
                pltpu.async_copy(table_hbm.at[src_v.at[jn]],
                                 bufs[1 - b], sems[1 - b])
                pltpu.make_async_copy(table_hbm.at[src_v.at[j]],
                                      bufs[b], sems[b]).wait()
                pltpu.sync_copy(bufs[b], acc.at[dst_v.at[j]], add=True)
            return carry
        lax.fori_loop(0, n_chunks // 2, ebody, 0)
        pltpu.make_async_copy(table_hbm.at[src_v.at[last]], rows0, sem0).wait()
        plsc.subcore_barrier()

        def obody(i, carry):
            pltpu.sync_copy(acc.at[pl.ds(z0 + i * _CH, _CH)], rows0)
            pltpu.sync_copy(rows0, out_hbm.at[c, pl.ds(z0 + i * _CH, _CH)])
            return carry
        lax.fori_loop(0, zit, obody, 0)

    return k


_seg_edge = _seg_sum(_NA)
_seg_feat = _seg_sum(_NB)


def _tc1(parts, x_aug, wl, wr, b):
    def body(p_ref, x_ref, wl_ref, wr_ref, b_ref, h_ref, inv_ref):
        sums = p_ref[0] + p_ref[1]
        inv = 1.0 / jnp.maximum(sums[:, 5:6], 1.0)
        h = (sums * inv) @ wl_ref[...] + x_ref[...] @ wr_ref[...] + b_ref[...]
        h = jnp.maximum(h, 0.0)
        for q in range(8):
            h_ref[q] = h[:, 8 * q:8 * (q + 1)]
        inv_ref[...] = inv

    return pl.pallas_call(
        body,
        grid=(_N // _BN,),
        in_specs=[
            pl.BlockSpec((2, _BN, 8), lambda i: (0, i, 0)),
            pl.BlockSpec((_BN, 8), lambda i: (i, 0)),
            pl.BlockSpec((8, 64), lambda i: (0, 0)),
            pl.BlockSpec((8, 64), lambda i: (0, 0)),
            pl.BlockSpec((1, 64), lambda i: (0, 0)),
        ],
        out_specs=[
            pl.BlockSpec((8, _BN, 8), lambda i: (0, i, 0)),
            pl.BlockSpec((_BN, 1), lambda i: (i, 0)),
        ],
        out_shape=[
            jax.ShapeDtypeStruct((8, _N, 8), jnp.float32),
            jax.ShapeDtypeStruct((_N, 1), jnp.float32),
        ],
    )(parts, x_aug, wl, wr, b)


def _tc2(a0, a1, a2, a3, h_split, inv, wl2, wr2, b2, wl3, wr3, b3):
    def body(a0_ref, a1_ref, a2_ref, a3_ref, h_ref, inv_ref,
             wl2_ref, wr2_ref, b2_ref, wl3_ref, wr3_ref, b3_ref,
             p_ref, r_ref):
        blocks = []
        for a_ref in (a0_ref, a1_ref, a2_ref, a3_ref):
            blocks.append(a_ref[0])
            blocks.append(a_ref[1])
        agg = jnp.concatenate(blocks, axis=1) * inv_ref[...]
        h1 = jnp.concatenate([h_ref[q] for q in range(8)], axis=1)
        h2 = agg @ wl2_ref[...] + h1 @ wr2_ref[...] + b2_ref[...]
        h2 = jnp.maximum(h2, 0.0)
        p = h2 @ wl3_ref[...]
        for q in range(4):
            p_ref[q] = p[:, 8 * q:8 * (q + 1)]
        r_ref[...] = h2 @ wr3_ref[...] + b3_ref[...]

    aspec = pl.BlockSpec((2, _BN, 8), lambda i: (0, i, 0))
    return pl.pallas_call(
        body,
        grid=(_N // _BN,),
        in_specs=[
            aspec, aspec, aspec, aspec,
            pl.BlockSpec((8, _BN, 8), lambda i: (0, i, 0)),
            pl.BlockSpec((_BN, 1), lambda i: (i, 0)),
            pl.BlockSpec((64, 64), lambda i: (0, 0)),
            pl.BlockSpec((64, 64), lambda i: (0, 0)),
            pl.BlockSpec((1, 64), lambda i: (0, 0)),
            pl.BlockSpec((64, 32), lambda i: (0, 0)),
            pl.BlockSpec((64, 32), lambda i: (0, 0)),
            pl.BlockSpec((1, 32), lambda i: (0, 0)),
        ],
        out_specs=[
            pl.BlockSpec((4, _BN, 8), lambda i: (0, i, 0)),
            pl.BlockSpec((_BN, 32), lambda i: (i, 0)),
        ],
        out_shape=[
            jax.ShapeDtypeStruct((4, _N, 8), jnp.float32),
            jax.ShapeDtypeStruct((_N, 32), jnp.float32),
        ],
    )(a0, a1, a2, a3, h_split, inv, wl2, wr2, b2, wl3, wr3, b3)


def _tc3(g0, g1, inv, r):
    def body(g0_ref, g1_ref, inv_ref, r_ref, out_ref):
        agg = jnp.concatenate([g0_ref[0], g0_ref[1], g1_ref[0], g1_ref[1]],
                              axis=1)
        out_ref[...] = agg * inv_ref[...] + r_ref[...]

    gspec = pl.BlockSpec((2, _BN, 8), lambda i: (0, i, 0))
    return pl.pallas_call(
        body,
        grid=(_N // _BN,),
        in_specs=[
            gspec, gspec,
            pl.BlockSpec((_BN, 1), lambda i: (i, 0)),
            pl.BlockSpec((_BN, 32), lambda i: (i, 0)),
        ],
        out_specs=pl.BlockSpec((_BN, 32), lambda i: (i, 0)),
        out_shape=jax.ShapeDtypeStruct((_N, 32), jnp.float32),
    )(g0, g1, inv, r)


def kernel(x, edge_index, Wl1, Wr1, b1, Wl2, Wr2, b2, Wl3, Wr3, b3):
    src = edge_index[0]
    dst = edge_index[1]
    pad = _EPAD - _E
    srcp = jnp.concatenate([src, jnp.zeros((pad,), jnp.int32)])
    dstp = jnp.concatenate([dst, jnp.full((pad,), _N, jnp.int32)])

    srcA = srcp.reshape(_NC, _NS, _NA, _CH)
    dstA = dstp.reshape(_NC, _NS, _NA, _CH)
    srcS = srcp.reshape(1, _NS, _NB, _CH)
    srcF = [jnp.concatenate([srcS + (2 * q) * _N, srcS + (2 * q + 1) * _N])
            for q in range(4)]
    dstB = jnp.broadcast_to(dstp.reshape(1, _NS, _NB, _CH),
                            (_NC, _NS, _NB, _CH))

    ones = jnp.ones((_N, 1), jnp.float32)
    x_aug = jnp.concatenate([x, ones, jnp.zeros((_N, 2), jnp.float32)], axis=1)
    z8 = jnp.zeros((_CH, 8), jnp.float32)
    wpad = jnp.zeros((3, 64), jnp.float32)
    wl1p = jnp.concatenate([Wl1, wpad], axis=0)
    wr1p = jnp.concatenate([Wr1, wpad], axis=0)

    parts1 = _seg_edge(srcA, dstA, x_aug, z8)[:, :_N]
    h_split, inv = _tc1(parts1, x_aug, wl1p, wr1p, b1.reshape(1, 64))
    table2 = h_split.reshape(8 * _N, 8)
    agg2 = [_seg_feat(srcF[q], dstB, table2, z8)[:, :_N] for q in range(4)]
    p_split, r = _tc2(agg2[0], agg2[1], agg2[2], agg2[3], h_split, inv,
                      Wl2, Wr2, b2.reshape(1, 64), Wl3, Wr3, b3.reshape(1, 32))
    table3 = p_split.reshape(4 * _N, 8)
    agg3 = [_seg_feat(srcF[q], dstB, table3, z8)[:, :_N] for q in range(2)]
    return _tc3(agg3[0], agg3[1], inv, r)


# speedup vs baseline: 6.5084x; 1.2214x over previous
import functools

import jax
import jax.numpy as jnp
from jax import lax
from jax.experimental import pallas as pl
from jax.experimental.pallas import tpu as pltpu
from jax.experimental.pallas import tpu_sc as plsc

_N = 50000
_E = 800000
_NC = 2
_NS = 16
_CH = 128
_EPAD = 802816
_NA = 196
_NB = 392
_NACC = 51200
_BN = 2000


def _seg_sum(n_chunks, edge_split):
    mesh = plsc.VectorSubcoreMesh(core_axis_name="c", subcore_axis_name="s")
    zit = (_NACC // _NS) // _CH

    scratch = [
        pltpu.VMEM((n_chunks, _CH), jnp.int32),
        pltpu.VMEM((n_chunks, _CH), jnp.int32),
        pltpu.VMEM((_CH, 8), jnp.float32),
        pltpu.VMEM((_CH, 8), jnp.float32),
    ]
    if not edge_split:
        scratch.append(pltpu.VMEM((2, 16), jnp.int32))
    scratch += [
        pltpu.VMEM_SHARED((_NACC, 8), jnp.float32),
        pltpu.SemaphoreType.DMA,
        pltpu.SemaphoreType.DMA,
    ]

    def body(src_hbm, dst_hbm, table_hbm, zeros_hbm, moff_hbm, out_hbm,
             src_v, dst_v, rows0, rows1, moff_v, acc, sem0, sem1):
        c = lax.axis_index("c")
        s = lax.axis_index("s")

        pltpu.sync_copy(zeros_hbm, rows0)
        z0 = s * (_NACC // _NS)

        def zbody(i, carry):
            pltpu.sync_copy(rows0, acc.at[pl.ds(z0 + i * _CH, _CH)])
            return carry
        lax.fori_loop(0, zit, zbody, 0)
        plsc.subcore_barrier()

        if edge_split:
            w = c * _NS + s
            pltpu.sync_copy(src_hbm.at[w], src_v)
            pltpu.sync_copy(dst_hbm.at[w], dst_v)
        else:
            pltpu.sync_copy(src_hbm.at[s], src_v)
            pltpu.sync_copy(dst_hbm.at[s], dst_v)
            pltpu.sync_copy(moff_hbm.at[c], moff_v)
            m = moff_v[0]
            o = moff_v[1]

            def tbody(j, carry):
                for k in range(_CH // 16):
                    sl = (j, pl.ds(16 * k, 16))
                    src_v[sl] = src_v[sl] * m + o
                return carry
            lax.fori_loop(0, n_chunks, tbody, 0)

        bufs = (rows0, rows1)
        sems = (sem0, sem1)
        last = n_chunks - 1

        pltpu.async_copy(table_hbm.at[src_v.at[0]], rows0, sem0)

        def ebody(t, carry):
            for b in range(2):
                j = 2 * t + b
                jn = jnp.minimum(j + 1, last)
                pltpu.async_copy(table_hbm.at[src_v.at[jn]],
                                 bufs[1 - b], sems[1 - b])
                pltpu.make_async_copy(table_hbm.at[src_v.at[j]],
                                      bufs[b], sems[b]).wait()
                pltpu.sync_copy(bufs[b], acc.at[dst_v.at[j]], add=True)
            return carry
        lax.fori_loop(0, n_chunks // 2, ebody, 0)
        pltpu.make_async_copy(table_hbm.at[src_v.at[last]], rows0, sem0).wait()
        plsc.subcore_barrier()

        def obody(i, carry):
            pltpu.sync_copy(acc.at[pl.ds(z0 + i * _CH, _CH)], rows0)
            pltpu.sync_copy(rows0, out_hbm.at[c, pl.ds(z0 + i * _CH, _CH)])
            return carry
        lax.fori_loop(0, zit, obody, 0)

    if edge_split:
        def body_e(src_hbm, dst_hbm, table_hbm, zeros_hbm, out_hbm,
                   src_v, dst_v, rows0, rows1, acc, sem0, sem1):
            body(src_hbm, dst_hbm, table_hbm, zeros_hbm, None, out_hbm,
                 src_v, dst_v, rows0, rows1, None, acc, sem0, sem1)
        fn = body_e
    else:
        fn = body

    return pl.kernel(
        fn,
        out_type=jax.ShapeDtypeStruct((_NC, _NACC, 8), jnp.float32),
        mesh=mesh,
        scratch_types=scratch,
        compiler_params=pltpu.CompilerParams(use_tc_tiling_on_sc=False),
    )


_seg_edge = _seg_sum(_NA, True)
_seg_feat = _seg_sum(_NB, False)


def _tc1(parts, x_aug, wl, wr, b):
    def body(p_ref, x_ref, wl_ref, wr_ref, b_ref, h_ref, inv_ref):
        sums = p_ref[0] + p_ref[1]
        inv = 1.0 / jnp.maximum(sums[:, 5:6], 1.0)
        h = (sums * inv) @ wl_ref[...] + x_ref[...] @ wr_ref[...] + b_ref[...]
        h_ref[...] = jnp.maximum(h, 0.0)
        inv_ref[...] = inv

    return pl.pallas_call(
        body,
        grid=(_N // _BN,),
        in_specs=[
            pl.BlockSpec((2, _BN, 8), lambda i: (0, i, 0)),
            pl.BlockSpec((_BN, 8), lambda i: (i, 0)),
            pl.BlockSpec((8, 64), lambda i: (0, 0)),
            pl.BlockSpec((8, 64), lambda i: (0, 0)),
            pl.BlockSpec((1, 64), lambda i: (0, 0)),
        ],
        out_specs=[
            pl.BlockSpec((_BN, 64), lambda i: (i, 0)),
            pl.BlockSpec((_BN, 1), lambda i: (i, 0)),
        ],
        out_shape=[
            jax.ShapeDtypeStruct((_N, 64), jnp.float32),
            jax.ShapeDtypeStruct((_N, 1), jnp.float32),
        ],
    )(parts, x_aug, wl, wr, b)


def _tc2(a0, a1, a2, a3, h, inv, wl2, wr2, b2, wl3, wr3, b3):
    def body(a0_ref, a1_ref, a2_ref, a3_ref, h_ref, inv_ref,
             wl2_ref, wr2_ref, b2_ref, wl3_ref, wr3_ref, b3_ref,
             p_ref, r_ref):
        blocks = []
        for a_ref in (a0_ref, a1_ref, a2_ref, a3_ref):
            blocks.append(a_ref[0])
            blocks.append(a_ref[1])
        agg = jnp.concatenate(blocks, axis=1) * inv_ref[...]
        h2 = agg @ wl2_ref[...] + h_ref[...] @ wr2_ref[...] + b2_ref[...]
        h2 = jnp.maximum(h2, 0.0)
        p_ref[...] = h2 @ wl3_ref[...]
        r_ref[...] = h2 @ wr3_ref[...] + b3_ref[...]

    aspec = pl.BlockSpec((2, _BN, 8), lambda i: (0, i, 0))
    return pl.pallas_call(
        body,
        grid=(_N // _BN,),
        in_specs=[
            aspec, aspec, aspec, aspec,
            pl.BlockSpec((_BN, 64), lambda i: (i, 0)),
            pl.BlockSpec((_BN, 1), lambda i: (i, 0)),
            pl.BlockSpec((64, 64), lambda i: (0, 0)),
            pl.BlockSpec((64, 64), lambda i: (0, 0)),
            pl.BlockSpec((1, 64), lambda i: (0, 0)),
            pl.BlockSpec((64, 32), lambda i: (0, 0)),
            pl.BlockSpec((64, 32), lambda i: (0, 0)),
            pl.BlockSpec((1, 32), lambda i: (0, 0)),
        ],
        out_specs=[
            pl.BlockSpec((_BN, 32), lambda i: (i, 0)),
            pl.BlockSpec((_BN, 32), lambda i: (i, 0)),
        ],
        out_shape=[
            jax.ShapeDtypeStruct((_N, 32), jnp.float32),
            jax.ShapeDtypeStruct((_N, 32), jnp.float32),
        ],
    )(a0, a1, a2, a3, h, inv, wl2, wr2, b2, wl3, wr3, b3)


def _tc3(g0, g1, inv, r):
    def body(g0_ref, g1_ref, inv_ref, r_ref, out_ref):
        agg = jnp.concatenate([g0_ref[0], g0_ref[1], g1_ref[0], g1_ref[1]],
                              axis=1)
        out_ref[...] = agg * inv_ref[...] + r_ref[...]

    gspec = pl.BlockSpec((2, _BN, 8), lambda i: (0, i, 0))
    return pl.pallas_call(
        body,
        grid=(_N // _BN,),
        in_specs=[
            gspec, gspec,
            pl.BlockSpec((_BN, 1), lambda i: (i, 0)),
            pl.BlockSpec((_BN, 32), lambda i: (i, 0)),
        ],
        out_specs=pl.BlockSpec((_BN, 32), lambda i: (i, 0)),
        out_shape=jax.ShapeDtypeStruct((_N, 32), jnp.float32),
    )(g0, g1, inv, r)


def _moff(mul, q):
    def one(off):
        return jnp.stack([jnp.full((16,), mul, jnp.int32),
                          jnp.full((16,), off, jnp.int32)])
    return jnp.stack([one(2 * q), one(2 * q + 1)])


def kernel(x, edge_index, Wl1, Wr1, b1, Wl2, Wr2, b2, Wl3, Wr3, b3):
    src = edge_index[0]
    dst = edge_index[1]
    pad = _EPAD - _E
    srcp = jnp.concatenate([src, jnp.zeros((pad,), jnp.int32)])
    dstp = jnp.concatenate([dst, jnp.full((pad,), _N, jnp.int32)])

    srcA = srcp.reshape(_NC * _NS, _NA, _CH)
    dstA = dstp.reshape(_NC * _NS, _NA, _CH)
    srcB = srcp.reshape(_NS, _NB, _CH)
    dstB = dstp.reshape(_NS, _NB, _CH)

    ones = jnp.ones((_N, 1), jnp.float32)
    x_aug = jnp.concatenate([x, ones, jnp.zeros((_N, 2), jnp.float32)], axis=1)
    z8 = jnp.zeros((_CH, 8), jnp.float32)
    wpad = jnp.zeros((3, 64), jnp.float32)
    wl1p = jnp.concatenate([Wl1, wpad], axis=0)
    wr1p = jnp.concatenate([Wr1, wpad], axis=0)

    parts1 = _seg_edge(srcA, dstA, x_aug, z8)
    h, inv = _tc1(parts1, x_aug, wl1p, wr1p, b1.reshape(1, 64))
    table2 = h.reshape(8 * _N, 8)
    agg2 = [_seg_feat(srcB, dstB, table2, z8, _moff(8, q)) for q in range(4)]
    p, r = _tc2(agg2[0], agg2[1], agg2[2], agg2[3], h, inv,
                Wl2, Wr2, b2.reshape(1, 64), Wl3, Wr3, b3.reshape(1, 32))
    table3 = p.reshape(4 * _N, 8)
    agg3 = [_seg_feat(srcB, dstB, table3, z8, _moff(4, q)) for q in range(2)]
    return _tc3(agg3[0], agg3[1], inv, r)


# speedup vs baseline: 7.9803x; 1.2262x over previous
import functools

import jax
import jax.numpy as jnp
from jax import lax
from jax.experimental import pallas as pl
from jax.experimental.pallas import tpu as pltpu
from jax.experimental.pallas import tpu_sc as plsc

_N = 50000
_E = 800000
_NC = 2
_NS = 16
_CH = 256
_EPAD = 802816
_NA = _EPAD // (_NC * _NS * _CH)
_NB = _EPAD // (_NS * _CH)
_NACC = 50176
_ZCH = 64
_BN = 2000


def _seg_sum(n_chunks, edge_split):
    mesh = plsc.VectorSubcoreMesh(core_axis_name="c", subcore_axis_name="s")
    zit = (_NACC // _NS) // _ZCH

    scratch = [
        pltpu.VMEM((n_chunks, _CH), jnp.int32),
        pltpu.VMEM((n_chunks, _CH), jnp.int32),
        pltpu.VMEM((_CH, 8), jnp.float32),
        pltpu.VMEM((_CH, 8), jnp.float32),
        pltpu.VMEM((_ZCH, 8), jnp.float32),
    ]
    if not edge_split:
        scratch.append(pltpu.VMEM((2, 16), jnp.int32))
    scratch += [
        pltpu.VMEM_SHARED((_NACC, 8), jnp.float32),
        pltpu.SemaphoreType.DMA,
        pltpu.SemaphoreType.DMA,
    ]

    def body(src_hbm, dst_hbm, table_hbm, zeros_hbm, moff_hbm, out_hbm,
             src_v, dst_v, rows0, rows1, zbuf, moff_v, acc, sem0, sem1):
        c = lax.axis_index("c")
        s = lax.axis_index("s")

        pltpu.sync_copy(zeros_hbm, zbuf)
        z0 = s * (_NACC // _NS)

        def zbody(i, carry):
            pltpu.sync_copy(zbuf, acc.at[pl.ds(z0 + i * _ZCH, _ZCH)])
            return carry
        lax.fori_loop(0, zit, zbody, 0)
        plsc.subcore_barrier()

        if edge_split:
            w = c * _NS + s
            pltpu.sync_copy(src_hbm.at[w], src_v)
            pltpu.sync_copy(dst_hbm.at[w], dst_v)
        else:
            pltpu.sync_copy(src_hbm.at[s], src_v)
            pltpu.sync_copy(dst_hbm.at[s], dst_v)
            pltpu.sync_copy(moff_hbm.at[c], moff_v)
            m = moff_v[0]
            o = moff_v[1]

            def tbody(j, carry):
                for k in range(_CH // 16):
                    sl = (j, pl.ds(16 * k, 16))
                    src_v[sl] = src_v[sl] * m + o
                return carry
            lax.fori_loop(0, n_chunks, tbody, 0)

        bufs = (rows0, rows1)
        sems = (sem0, sem1)
        last = n_chunks - 1

        pltpu.async_copy(table_hbm.at[src_v.at[0]], rows0, sem0)

        def ebody(t, carry):
            for b in range(2):
                j = 2 * t + b
                jn = jnp.minimum(j + 1, last)
                pltpu.async_copy(table_hbm.at[src_v.at[jn]],
                                 bufs[1 - b], sems[1 - b])
                pltpu.make_async_copy(table_hbm.at[src_v.at[j]],
                                      bufs[b], sems[b]).wait()
                pltpu.sync_copy(bufs[b], acc.at[dst_v.at[j]], add=True)
            return carry
        lax.fori_loop(0, n_chunks // 2, ebody, 0)
        pltpu.make_async_copy(table_hbm.at[src_v.at[last]], rows0, sem0).wait()
        plsc.subcore_barrier()

        def obody(i, carry):
            pltpu.sync_copy(acc.at[pl.ds(z0 + i * _ZCH, _ZCH)], zbuf)
            pltpu.sync_copy(zbuf, out_hbm.at[c, pl.ds(z0 + i * _ZCH, _ZCH)])
            return carry
        lax.fori_loop(0, zit, obody, 0)

    if edge_split:
        def body_e(src_hbm, dst_hbm, table_hbm, zeros_hbm, out_hbm,
                   src_v, dst_v, rows0, rows1, zbuf, acc, sem0, sem1):
            body(src_hbm, dst_hbm, table_hbm, zeros_hbm, None, out_hbm,
                 src_v, dst_v, rows0, rows1, zbuf, None, acc, sem0, sem1)
        fn = body_e
    else:
        fn = body

    return pl.kernel(
        fn,
        out_type=jax.ShapeDtypeStruct((_NC, _NACC, 8), jnp.float32),
        mesh=mesh,
        scratch_types=scratch,
        compiler_params=pltpu.CompilerParams(use_tc_tiling_on_sc=False),
    )


_seg_edge = _seg_sum(_NA, True)
_seg_feat = _seg_sum(_NB, False)


def _tc1(parts, x_aug, wl, wr, b):
    def body(p_ref, x_ref, wl_ref, wr_ref, b_ref, h_ref, inv_ref):
        sums = p_ref[0] + p_ref[1]
        inv = 1.0 / jnp.maximum(sums[:, 5:6], 1.0)
        h = (sums * inv) @ wl_ref[...] + x_ref[...] @ wr_ref[...] + b_ref[...]
        h_ref[...] = jnp.maximum(h, 0.0)
        inv_ref[...] = inv

    return pl.pallas_call(
        body,
        grid=(_N // _BN,),
        in_specs=[
            pl.BlockSpec((2, _BN, 8), lambda i: (0, i, 0)),
            pl.BlockSpec((_BN, 8), lambda i: (i, 0)),
            pl.BlockSpec((8, 64), lambda i: (0, 0)),
            pl.BlockSpec((8, 64), lambda i: (0, 0)),
            pl.BlockSpec((1, 64), lambda i: (0, 0)),
        ],
        out_specs=[
            pl.BlockSpec((_BN, 64), lambda i: (i, 0)),
            pl.BlockSpec((_BN, 1), lambda i: (i, 0)),
        ],
        out_shape=[
            jax.ShapeDtypeStruct((_N, 64), jnp.float32),
            jax.ShapeDtypeStruct((_N, 1), jnp.float32),
        ],
    )(parts, x_aug, wl, wr, b)


def _tc2(a0, a1, a2, a3, h, inv, wl2, wr2, b2, wl3, wr3, b3):
    def body(a0_ref, a1_ref, a2_ref, a3_ref, h_ref, inv_ref,
             wl2_ref, wr2_ref, b2_ref, wl3_ref, wr3_ref, b3_ref,
             p_ref, r_ref):
        blocks = []
        for a_ref in (a0_ref, a1_ref, a2_ref, a3_ref):
            blocks.append(a_ref[0])
            blocks.append(a_ref[1])
        agg = jnp.concatenate(blocks, axis=1) * inv_ref[...]
        h2 = agg @ wl2_ref[...] + h_ref[...] @ wr2_ref[...] + b2_ref[...]
        h2 = jnp.maximum(h2, 0.0)
        p_ref[...] = h2 @ wl3_ref[...]
        r_ref[...] = h2 @ wr3_ref[...] + b3_ref[...]

    aspec = pl.BlockSpec((2, _BN, 8), lambda i: (0, i, 0))
    return pl.pallas_call(
        body,
        grid=(_N // _BN,),
        in_specs=[
            aspec, aspec, aspec, aspec,
            pl.BlockSpec((_BN, 64), lambda i: (i, 0)),
            pl.BlockSpec((_BN, 1), lambda i: (i, 0)),
            pl.BlockSpec((64, 64), lambda i: (0, 0)),
            pl.BlockSpec((64, 64), lambda i: (0, 0)),
            pl.BlockSpec((1, 64), lambda i: (0, 0)),
            pl.BlockSpec((64, 32), lambda i: (0, 0)),
            pl.BlockSpec((64, 32), lambda i: (0, 0)),
            pl.BlockSpec((1, 32), lambda i: (0, 0)),
        ],
        out_specs=[
            pl.BlockSpec((_BN, 32), lambda i: (i, 0)),
            pl.BlockSpec((_BN, 32), lambda i: (i, 0)),
        ],
        out_shape=[
            jax.ShapeDtypeStruct((_N, 32), jnp.float32),
            jax.ShapeDtypeStruct((_N, 32), jnp.float32),
        ],
    )(a0, a1, a2, a3, h, inv, wl2, wr2, b2, wl3, wr3, b3)


def _tc3(g0, g1, inv, r):
    def body(g0_ref, g1_ref, inv_ref, r_ref, out_ref):
        agg = jnp.concatenate([g0_ref[0], g0_ref[1], g1_ref[0], g1_ref[1]],
                              axis=1)
        out_ref[...] = agg * inv_ref[...] + r_ref[...]

    gspec = pl.BlockSpec((2, _BN, 8), lambda i: (0, i, 0))
    return pl.pallas_call(
        body,
        grid=(_N // _BN,),
        in_specs=[
            gspec, gspec,
            pl.BlockSpec((_BN, 1), lambda i: (i, 0)),
            pl.BlockSpec((_BN, 32), lambda i: (i, 0)),
        ],
        out_specs=pl.BlockSpec((_BN, 32), lambda i: (i, 0)),
        out_shape=jax.ShapeDtypeStruct((_N, 32), jnp.float32),
    )(g0, g1, inv, r)


def _moff(mul, q):
    def one(off):
        return jnp.stack([jnp.full((16,), mul, jnp.int32),
                          jnp.full((16,), off, jnp.int32)])
    return jnp.stack([one(2 * q), one(2 * q + 1)])


def kernel(x, edge_index, Wl1, Wr1, b1, Wl2, Wr2, b2, Wl3, Wr3, b3):
    src = edge_index[0]
    dst = edge_index[1]
    pad = _EPAD - _E
    srcp = jnp.concatenate([src, jnp.zeros((pad,), jnp.int32)])
    dstp = jnp.concatenate([dst, jnp.full((pad,), _N, jnp.int32)])

    srcA = srcp.reshape(_NC * _NS, _NA, _CH)
    dstA = dstp.reshape(_NC * _NS, _NA, _CH)
    srcB = srcp.reshape(_NS, _NB, _CH)
    dstB = dstp.reshape(_NS, _NB, _CH)

    ones = jnp.ones((_N, 1), jnp.float32)
    x_aug = jnp.concatenate([x, ones, jnp.zeros((_N, 2), jnp.float32)], axis=1)
    z8 = jnp.zeros((_ZCH, 8), jnp.float32)
    wpad = jnp.zeros((3, 64), jnp.float32)
    wl1p = jnp.concatenate([Wl1, wpad], axis=0)
    wr1p = jnp.concatenate([Wr1, wpad], axis=0)

    parts1 = _seg_edge(srcA, dstA, x_aug, z8)
    h, inv = _tc1(parts1, x_aug, wl1p, wr1p, b1.reshape(1, 64))
    table2 = h.reshape(8 * _N, 8)
    agg2 = [_seg_feat(srcB, dstB, table2, z8, _moff(8, q)) for q in range(4)]
    p, r = _tc2(agg2[0], agg2[1], agg2[2], agg2[3], h, inv,
                Wl2, Wr2, b2.reshape(1, 64), Wl3, Wr3, b3.reshape(1, 32))
    table3 = p.reshape(4 * _N, 8)
    agg3 = [_seg_feat(srcB, dstB, table3, z8, _moff(4, q)) for q in range(2)]
    return _tc3(agg3[0], agg3[1], inv, r)
